# baseline (speedup 1.0000x reference)
import functools

import jax
import jax.numpy as jnp
from jax import lax
from jax.experimental import pallas as pl
from jax.experimental.pallas import tpu as pltpu
from jax.experimental.pallas import tpu_sc as plsc

N = 10000
E = 320000
D = 128
NPRIM = 4

NC = 2
NS = 16
NW = NC * NS
E_PER_TILE = E // NW
CHUNK = 80
N_CHUNKS = E_PER_TILE // CHUNK
N_PAD = 10240
ROWS_PER_TILE = N_PAD // NS

BR = 2000


def _mm_body(a_ref, W_ref, x_ref, h_ref):
    a = a_ref[...]
    e = jnp.exp(a - jnp.max(a))
    w = e / jnp.sum(e)
    W_eff = (W_ref[0] * w[0:1, 0:1] + W_ref[1] * w[0:1, 1:2]
             + W_ref[2] * w[0:1, 2:3] + W_ref[3] * w[0:1, 3:4])
    h_ref[...] = jnp.dot(x_ref[...], W_eff,
                         preferred_element_type=jnp.float32)


_mm_call = pl.pallas_call(
    _mm_body,
    grid=(N // BR,),
    in_specs=[
        pl.BlockSpec((1, NPRIM), lambda i: (0, 0)),
        pl.BlockSpec((NPRIM, D, D), lambda i: (0, 0, 0)),
        pl.BlockSpec((BR, D), lambda i: (i, 0)),
    ],
    out_specs=pl.BlockSpec((BR, D), lambda i: (i, 0)),
    out_shape=jax.ShapeDtypeStruct((N, D), jnp.float32),
)


SEC = 25
N_SEC = N_CHUNKS // SEC


def _sc_body(h_hbm, ei_hbm, zeros_hbm, out_hbm,
             src_v, dst_v, rows_v, acc_sh, semG0, semG1, semG2,
             semS0, semS1, semS2, semI):
    cid = lax.axis_index("c")
    sid = lax.axis_index("s")
    wid = cid * NS + sid
    r0 = sid * ROWS_PER_TILE
    semG = (semG0, semG1, semG2)
    semS = (semS0, semS1, semS2)

    pltpu.sync_copy(zeros_hbm, acc_sh.at[pl.ds(r0, ROWS_PER_TILE)])
    plsc.subcore_barrier()

    base = wid * E_PER_TILE
    pltpu.sync_copy(ei_hbm.at[pl.ds(base, SEC * CHUNK)], src_v.at[pl.ds(0, SEC * CHUNK)])
    pltpu.sync_copy(ei_hbm.at[pl.ds(E + base, SEC * CHUNK)], dst_v.at[pl.ds(0, SEC * CHUNK)])
    H = CHUNK // 2

    def _gather(off, r, sem):
        pltpu.async_copy(h_hbm.at[src_v.at[pl.ds(off, H)]],
                         rows_v.at[r, pl.ds(0, H)], sem)
        pltpu.async_copy(h_hbm.at[src_v.at[pl.ds(off + H, H)]],
                         rows_v.at[r, pl.ds(H, H)], sem)

    def _scatter(off, r, sem):
        pltpu.async_copy(rows_v.at[r, pl.ds(0, H)],
                         acc_sh.at[dst_v.at[pl.ds(off, H)]], sem, add=True)
        pltpu.async_copy(rows_v.at[r, pl.ds(H, H)],
                         acc_sh.at[dst_v.at[pl.ds(off + H, H)]], sem, add=True)

    _gather(0, 0, semG0)
    _gather(CHUNK, 1, semG1)
    pltpu.make_async_copy(h_hbm.at[pl.ds(0, CHUNK)], rows_v.at[0], semG0).wait()
    _scatter(0, 0, semS0)
    _gather(2 * CHUNK, 2, semG2)
    pltpu.make_async_copy(h_hbm.at[pl.ds(0, CHUNK)], rows_v.at[1], semG1).wait()
    _scatter(CHUNK, 1, semS1)

    def body(c, carry):
        s = c // SEC
        cm = lax.rem(c, SEC)
        b = lax.rem(s, 2)
        g = lax.min(c + 1, N_CHUNKS - 1)
        s1 = g // SEC
        b1 = lax.rem(s1, 2)
        row1 = lax.rem(g, SEC)

        for k in range(3):
            @pl.when(lax.rem(c, 3) == k)
            def _(k=k):
                r, r1 = k, (k + 1) % 3

                @pl.when(cm == SEC - 1)
                def _():
                    pltpu.make_async_copy(ei_hbm.at[pl.ds(0, SEC * CHUNK)],
                                          src_v.at[pl.ds(0, SEC * CHUNK)],
                                          semI).wait()
                    pltpu.make_async_copy(ei_hbm.at[pl.ds(E, SEC * CHUNK)],
                                          dst_v.at[pl.ds(0, SEC * CHUNK)],
                                          semI).wait()

                pltpu.make_async_copy(h_hbm.at[pl.ds(0, CHUNK)],
                                      rows_v.at[r1], semS[r1]).wait()
                _gather(b1 * SEC * CHUNK + row1 * CHUNK, r1, semG[r1])
                pltpu.make_async_copy(h_hbm.at[pl.ds(0, CHUNK)],
                                      rows_v.at[r], semG[r]).wait()
                _scatter(b * SEC * CHUNK + cm * CHUNK, r, semS[r])

                @pl.when(cm == 2)
                def _():
                    ns = lax.min(s + 1, N_SEC - 1)
                    nb = lax.rem(s + 1, 2)
                    off = base + ns * SEC * CHUNK
                    voff = nb * SEC * CHUNK
                    pltpu.async_copy(ei_hbm.at[pl.ds(off, SEC * CHUNK)],
                                     src_v.at[pl.ds(voff, SEC * CHUNK)], semI)
                    pltpu.async_copy(ei_hbm.at[pl.ds(E + off, SEC * CHUNK)],
                                     dst_v.at[pl.ds(voff, SEC * CHUNK)], semI)

        return carry

    lax.fori_loop(2, N_CHUNKS, body, 0)
    pltpu.make_async_copy(h_hbm.at[pl.ds(0, CHUNK)], rows_v.at[2], semG2).wait()
    pltpu.make_async_copy(h_hbm.at[pl.ds(0, CHUNK)], rows_v.at[0], semS0).wait()
    pltpu.make_async_copy(h_hbm.at[pl.ds(0, CHUNK)], rows_v.at[1], semS1).wait()
    plsc.subcore_barrier()

    pltpu.sync_copy(acc_sh.at[pl.ds(r0, ROWS_PER_TILE)],
                    out_hbm.at[cid, pl.ds(r0, ROWS_PER_TILE)])


_sc_call = functools.partial(
    pl.kernel,
    out_type=jax.ShapeDtypeStruct((NC, N_PAD, D), jnp.float32),
    mesh=plsc.VectorSubcoreMesh(core_axis_name="c", subcore_axis_name="s"),
    scratch_types=[
        pltpu.VMEM((2 * SEC * CHUNK,), jnp.int32),
        pltpu.VMEM((2 * SEC * CHUNK,), jnp.int32),
        pltpu.VMEM((3, CHUNK, D), jnp.float32),
        pltpu.VMEM_SHARED((N_PAD, D), jnp.float32),
        pltpu.SemaphoreType.DMA,
        pltpu.SemaphoreType.DMA,
        pltpu.SemaphoreType.DMA,
        pltpu.SemaphoreType.DMA,
        pltpu.SemaphoreType.DMA,
        pltpu.SemaphoreType.DMA,
        pltpu.SemaphoreType.DMA,
    ],
)(_sc_body)


def _comb_body(p_ref, m_ref, o_ref):
    o_ref[...] = m_ref[...] * (p_ref[0] + p_ref[1])


_comb_call = pl.pallas_call(
    _comb_body,
    grid=(N // BR,),
    in_specs=[
        pl.BlockSpec((NC, BR, D), lambda i: (0, i, 0)),
        pl.BlockSpec((BR, 1), lambda i: (i, 0)),
    ],
    out_specs=pl.BlockSpec((BR, D), lambda i: (i, 0)),
    out_shape=jax.ShapeDtypeStruct((N, D), jnp.float32),
)


def kernel(x, edge_index, mask, W, alphas):
    h = _mm_call(alphas.reshape(1, NPRIM), W, x)
    zeros = jnp.zeros((ROWS_PER_TILE, D), jnp.float32)
    partial = _sc_call(h, edge_index.reshape(-1), zeros)
    maskf = mask.astype(jnp.float32).reshape(N, 1)
    return _comb_call(partial, maskf)


# speedup vs baseline: 1.0261x; 1.0261x over previous
import functools

import jax
import jax.numpy as jnp
from jax import lax
from jax.experimental import pallas as pl
from jax.experimental.pallas import tpu as pltpu
from jax.experimental.pallas import tpu_sc as plsc

N = 10000
E = 320000
D = 128
NPRIM = 4

NC = 2
NS = 16
NW = NC * NS
E_PER_TILE = E // NW
CHUNK = 80
N_CHUNKS = E_PER_TILE // CHUNK
N_PAD = 10240
ROWS_PER_TILE = N_PAD // NS

BR = 5000


def _mm_body(a_ref, W_ref, x_ref, h_ref):
    a = a_ref[...]
    e = jnp.exp(a - jnp.max(a))
    w = e / jnp.sum(e)
    W_eff = (W_ref[0] * w[0:1, 0:1] + W_ref[1] * w[0:1, 1:2]
             + W_ref[2] * w[0:1, 2:3] + W_ref[3] * w[0:1, 3:4])
    h_ref[...] = jnp.dot(x_ref[...], W_eff,
                         preferred_element_type=jnp.float32)


_mm_call = pl.pallas_call(
    _mm_body,
    grid=(N // BR,),
    in_specs=[
        pl.BlockSpec((1, NPRIM), lambda i: (0, 0)),
        pl.BlockSpec((NPRIM, D, D), lambda i: (0, 0, 0)),
        pl.BlockSpec((BR, D), lambda i: (i, 0)),
    ],
    out_specs=pl.BlockSpec((BR, D), lambda i: (i, 0)),
    out_shape=jax.ShapeDtypeStruct((N, D), jnp.float32),
)


SEC = 25
N_SEC = N_CHUNKS // SEC


def _sc_body(h_hbm, ei_hbm, zeros_hbm, out_hbm,
             src_v, dst_v, rows_v, acc_sh, semG0, semG1, semG2,
             semS0, semS1, semS2, semI):
    cid = lax.axis_index("c")
    sid = lax.axis_index("s")
    wid = cid * NS + sid
    r0 = sid * ROWS_PER_TILE
    semG = (semG0, semG1, semG2)
    semS = (semS0, semS1, semS2)

    pltpu.sync_copy(zeros_hbm, acc_sh.at[pl.ds(r0, ROWS_PER_TILE)])
    plsc.subcore_barrier()

    base = wid * E_PER_TILE
    pltpu.sync_copy(ei_hbm.at[pl.ds(base, SEC * CHUNK)], src_v.at[pl.ds(0, SEC * CHUNK)])
    pltpu.sync_copy(ei_hbm.at[pl.ds(E + base, SEC * CHUNK)], dst_v.at[pl.ds(0, SEC * CHUNK)])
    H = CHUNK // 2

    def _gather(off, r, sem):
        pltpu.async_copy(h_hbm.at[src_v.at[pl.ds(off, H)]],
                         rows_v.at[r, pl.ds(0, H)], sem)
        pltpu.async_copy(h_hbm.at[src_v.at[pl.ds(off + H, H)]],
                         rows_v.at[r, pl.ds(H, H)], sem)

    def _scatter(off, r, sem):
        pltpu.async_copy(rows_v.at[r, pl.ds(0, H)],
                         acc_sh.at[dst_v.at[pl.ds(off, H)]], sem, add=True)
        pltpu.async_copy(rows_v.at[r, pl.ds(H, H)],
                         acc_sh.at[dst_v.at[pl.ds(off + H, H)]], sem, add=True)

    _gather(0, 0, semG0)
    _gather(CHUNK, 1, semG1)
    pltpu.make_async_copy(h_hbm.at[pl.ds(0, CHUNK)], rows_v.at[0], semG0).wait()
    _scatter(0, 0, semS0)
    _gather(2 * CHUNK, 2, semG2)
    pltpu.make_async_copy(h_hbm.at[pl.ds(0, CHUNK)], rows_v.at[1], semG1).wait()
    _scatter(CHUNK, 1, semS1)

    def body(c, carry):
        s = c // SEC
        cm = lax.rem(c, SEC)
        b = lax.rem(s, 2)
        g = lax.min(c + 1, N_CHUNKS - 1)
        s1 = g // SEC
        b1 = lax.rem(s1, 2)
        row1 = lax.rem(g, SEC)

        for k in range(3):
            @pl.when(lax.rem(c, 3) == k)
            def _(k=k):
                r, r1 = k, (k + 1) % 3

                @pl.when(cm == SEC - 1)
                def _():
                    pltpu.make_async_copy(ei_hbm.at[pl.ds(0, SEC * CHUNK)],
                                          src_v.at[pl.ds(0, SEC * CHUNK)],
                                          semI).wait()
                    pltpu.make_async_copy(ei_hbm.at[pl.ds(E, SEC * CHUNK)],
                                          dst_v.at[pl.ds(0, SEC * CHUNK)],
                                          semI).wait()

                pltpu.make_async_copy(h_hbm.at[pl.ds(0, CHUNK)],
                                      rows_v.at[r1], semS[r1]).wait()
                _gather(b1 * SEC * CHUNK + row1 * CHUNK, r1, semG[r1])
                pltpu.make_async_copy(h_hbm.at[pl.ds(0, CHUNK)],
                                      rows_v.at[r], semG[r]).wait()
                _scatter(b * SEC * CHUNK + cm * CHUNK, r, semS[r])

                @pl.when(cm == 2)
                def _():
                    ns = lax.min(s + 1, N_SEC - 1)
                    nb = lax.rem(s + 1, 2)
                    off = base + ns * SEC * CHUNK
                    voff = nb * SEC * CHUNK
                    pltpu.async_copy(ei_hbm.at[pl.ds(off, SEC * CHUNK)],
                                     src_v.at[pl.ds(voff, SEC * CHUNK)], semI)
                    pltpu.async_copy(ei_hbm.at[pl.ds(E + off, SEC * CHUNK)],
                                     dst_v.at[pl.ds(voff, SEC * CHUNK)], semI)

        return carry

    lax.fori_loop(2, N_CHUNKS, body, 0)
    pltpu.make_async_copy(h_hbm.at[pl.ds(0, CHUNK)], rows_v.at[2], semG2).wait()
    pltpu.make_async_copy(h_hbm.at[pl.ds(0, CHUNK)], rows_v.at[0], semS0).wait()
    pltpu.make_async_copy(h_hbm.at[pl.ds(0, CHUNK)], rows_v.at[1], semS1).wait()
    plsc.subcore_barrier()

    pltpu.sync_copy(acc_sh.at[pl.ds(r0, ROWS_PER_TILE)],
                    out_hbm.at[cid, pl.ds(r0, ROWS_PER_TILE)])


_sc_call = functools.partial(
    pl.kernel,
    out_type=jax.ShapeDtypeStruct((NC, N_PAD, D), jnp.float32),
    mesh=plsc.VectorSubcoreMesh(core_axis_name="c", subcore_axis_name="s"),
    scratch_types=[
        pltpu.VMEM((2 * SEC * CHUNK,), jnp.int32),
        pltpu.VMEM((2 * SEC * CHUNK,), jnp.int32),
        pltpu.VMEM((3, CHUNK, D), jnp.float32),
        pltpu.VMEM_SHARED((N_PAD, D), jnp.float32),
        pltpu.SemaphoreType.DMA,
        pltpu.SemaphoreType.DMA,
        pltpu.SemaphoreType.DMA,
        pltpu.SemaphoreType.DMA,
        pltpu.SemaphoreType.DMA,
        pltpu.SemaphoreType.DMA,
        pltpu.SemaphoreType.DMA,
    ],
)(_sc_body)


def _comb_body(p_ref, m_ref, o_ref):
    o_ref[...] = m_ref[...] * (p_ref[0] + p_ref[1])


_comb_call = pl.pallas_call(
    _comb_body,
    grid=(N // BR,),
    in_specs=[
        pl.BlockSpec((NC, BR, D), lambda i: (0, i, 0)),
        pl.BlockSpec((BR, 1), lambda i: (i, 0)),
    ],
    out_specs=pl.BlockSpec((BR, D), lambda i: (i, 0)),
    out_shape=jax.ShapeDtypeStruct((N, D), jnp.float32),
)


def kernel(x, edge_index, mask, W, alphas):
    h = _mm_call(alphas.reshape(1, NPRIM), W, x)
    zeros = jnp.zeros((ROWS_PER_TILE, D), jnp.float32)
    partial = _sc_call(h, edge_index.reshape(-1), zeros)
    maskf = mask.astype(jnp.float32).reshape(N, 1)
    return _comb_call(partial, maskf)


# speedup vs baseline: 1.0806x; 1.0531x over previous
import functools

import jax
import jax.numpy as jnp
from jax import lax
from jax.experimental import pallas as pl
from jax.experimental.pallas import tpu as pltpu
from jax.experimental.pallas import tpu_sc as plsc

N = 10000
E = 320000
D = 128
NPRIM = 4

NC = 2
NS = 16
NW = NC * NS
E_PER_TILE = E // NW
CHUNK = 80
N_CHUNKS = E_PER_TILE // CHUNK
N_PAD = 10240
ROWS_PER_TILE = N_PAD // NS

BR = 5000


def _mm_body(a_ref, W_ref, x_ref, h_ref):
    a = a_ref[...]
    e = jnp.exp(a - jnp.max(a))
    w = e / jnp.sum(e)
    W_eff = (W_ref[0] * w[0:1, 0:1] + W_ref[1] * w[0:1, 1:2]
             + W_ref[2] * w[0:1, 2:3] + W_ref[3] * w[0:1, 3:4])
    h_ref[...] = jnp.dot(x_ref[...], W_eff,
                         preferred_element_type=jnp.float32)


_mm_call = pl.pallas_call(
    _mm_body,
    grid=(N // BR,),
    in_specs=[
        pl.BlockSpec((1, NPRIM), lambda i: (0, 0)),
        pl.BlockSpec((NPRIM, D, D), lambda i: (0, 0, 0)),
        pl.BlockSpec((BR, D), lambda i: (i, 0)),
    ],
    out_specs=pl.BlockSpec((BR, D), lambda i: (i, 0)),
    out_shape=jax.ShapeDtypeStruct((N, D), jnp.float32),
)


SEC = 25
N_SEC = N_CHUNKS // SEC


def _sc_body(h_hbm, ei_hbm, out_hbm,
             src_v, dst_v, rows_v, acc_sh, semG0, semG1, semG2,
             semS0, semS1, semS2, semI):
    cid = lax.axis_index("c")
    sid = lax.axis_index("s")
    wid = cid * NS + sid
    r0 = sid * ROWS_PER_TILE
    semG = (semG0, semG1, semG2)
    semS = (semS0, semS1, semS2)

    def zi(j, carry):
        rows_v[0, j // 8, pl.ds(lax.rem(j, 8) * 16, 16)] = jnp.zeros(
            (16,), jnp.float32)
        return carry

    lax.fori_loop(0, CHUNK * 8, zi, 0)
    for j in range(ROWS_PER_TILE // CHUNK):
        pltpu.async_copy(rows_v.at[0], acc_sh.at[pl.ds(r0 + j * CHUNK, CHUNK)],
                         semI)
    pltpu.make_async_copy(h_hbm.at[pl.ds(0, ROWS_PER_TILE)],
                          acc_sh.at[pl.ds(r0, ROWS_PER_TILE)], semI).wait()
    plsc.subcore_barrier()

    base = wid * E_PER_TILE
    pltpu.sync_copy(ei_hbm.at[pl.ds(base, SEC * CHUNK)], src_v.at[pl.ds(0, SEC * CHUNK)])
    pltpu.sync_copy(ei_hbm.at[pl.ds(E + base, SEC * CHUNK)], dst_v.at[pl.ds(0, SEC * CHUNK)])
    H = CHUNK // 2

    def _gather(off, r, sem):
        pltpu.async_copy(h_hbm.at[src_v.at[pl.ds(off, H)]],
                         rows_v.at[r, pl.ds(0, H)], sem)
        pltpu.async_copy(h_hbm.at[src_v.at[pl.ds(off + H, H)]],
                         rows_v.at[r, pl.ds(H, H)], sem)

    def _scatter(off, r, sem):
        pltpu.async_copy(rows_v.at[r, pl.ds(0, H)],
                         acc_sh.at[dst_v.at[pl.ds(off, H)]], sem, add=True)
        pltpu.async_copy(rows_v.at[r, pl.ds(H, H)],
                         acc_sh.at[dst_v.at[pl.ds(off + H, H)]], sem, add=True)

    _gather(0, 0, semG0)
    _gather(CHUNK, 1, semG1)
    pltpu.make_async_copy(h_hbm.at[pl.ds(0, CHUNK)], rows_v.at[0], semG0).wait()
    _scatter(0, 0, semS0)
    _gather(2 * CHUNK, 2, semG2)
    pltpu.make_async_copy(h_hbm.at[pl.ds(0, CHUNK)], rows_v.at[1], semG1).wait()
    _scatter(CHUNK, 1, semS1)

    def body(c, carry):
        s = c // SEC
        cm = lax.rem(c, SEC)
        b = lax.rem(s, 2)
        g = lax.min(c + 1, N_CHUNKS - 1)
        s1 = g // SEC
        b1 = lax.rem(s1, 2)
        row1 = lax.rem(g, SEC)

        for k in range(3):
            @pl.when(lax.rem(c, 3) == k)
            def _(k=k):
                r, r1 = k, (k + 1) % 3

                @pl.when(cm == SEC - 1)
                def _():
                    pltpu.make_async_copy(ei_hbm.at[pl.ds(0, SEC * CHUNK)],
                                          src_v.at[pl.ds(0, SEC * CHUNK)],
                                          semI).wait()
                    pltpu.make_async_copy(ei_hbm.at[pl.ds(E, SEC * CHUNK)],
                                          dst_v.at[pl.ds(0, SEC * CHUNK)],
                                          semI).wait()

                pltpu.make_async_copy(h_hbm.at[pl.ds(0, CHUNK)],
                                      rows_v.at[r1], semS[r1]).wait()
                _gather(b1 * SEC * CHUNK + row1 * CHUNK, r1, semG[r1])
                pltpu.make_async_copy(h_hbm.at[pl.ds(0, CHUNK)],
                                      rows_v.at[r], semG[r]).wait()
                _scatter(b * SEC * CHUNK + cm * CHUNK, r, semS[r])

                @pl.when(cm == 2)
                def _():
                    ns = lax.min(s + 1, N_SEC - 1)
                    nb = lax.rem(s + 1, 2)
                    off = base + ns * SEC * CHUNK
                    voff = nb * SEC * CHUNK
                    pltpu.async_copy(ei_hbm.at[pl.ds(off, SEC * CHUNK)],
                                     src_v.at[pl.ds(voff, SEC * CHUNK)], semI)
                    pltpu.async_copy(ei_hbm.at[pl.ds(E + off, SEC * CHUNK)],
                                     dst_v.at[pl.ds(voff, SEC * CHUNK)], semI)

        return carry

    lax.fori_loop(2, N_CHUNKS, body, 0)
    pltpu.make_async_copy(h_hbm.at[pl.ds(0, CHUNK)], rows_v.at[2], semG2).wait()
    pltpu.make_async_copy(h_hbm.at[pl.ds(0, CHUNK)], rows_v.at[0], semS0).wait()
    pltpu.make_async_copy(h_hbm.at[pl.ds(0, CHUNK)], rows_v.at[1], semS1).wait()
    plsc.subcore_barrier()

    pltpu.sync_copy(acc_sh.at[pl.ds(r0, ROWS_PER_TILE)],
                    out_hbm.at[cid, pl.ds(r0, ROWS_PER_TILE)])


_sc_call = functools.partial(
    pl.kernel,
    out_type=jax.ShapeDtypeStruct((NC, N_PAD, D), jnp.float32),
    mesh=plsc.VectorSubcoreMesh(core_axis_name="c", subcore_axis_name="s"),
    scratch_types=[
        pltpu.VMEM((2 * SEC * CHUNK,), jnp.int32),
        pltpu.VMEM((2 * SEC * CHUNK,), jnp.int32),
        pltpu.VMEM((3, CHUNK, D), jnp.float32),
        pltpu.VMEM_SHARED((N_PAD, D), jnp.float32),
        pltpu.SemaphoreType.DMA,
        pltpu.SemaphoreType.DMA,
        pltpu.SemaphoreType.DMA,
        pltpu.SemaphoreType.DMA,
        pltpu.SemaphoreType.DMA,
        pltpu.SemaphoreType.DMA,
        pltpu.SemaphoreType.DMA,
    ],
)(_sc_body)


def _comb_body(p_ref, m_ref, o_ref):
    o_ref[...] = m_ref[...] * (p_ref[0] + p_ref[1])


_comb_call = pl.pallas_call(
    _comb_body,
    grid=(N // BR,),
    in_specs=[
        pl.BlockSpec((NC, BR, D), lambda i: (0, i, 0)),
        pl.BlockSpec((BR, 1), lambda i: (i, 0)),
    ],
    out_specs=pl.BlockSpec((BR, D), lambda i: (i, 0)),
    out_shape=jax.ShapeDtypeStruct((N, D), jnp.float32),
)


def kernel(x, edge_index, mask, W, alphas):
    h = _mm_call(alphas.reshape(1, NPRIM), W, x)
    partial = _sc_call(h, edge_index.reshape(-1))
    maskf = mask.astype(jnp.float32).reshape(N, 1)
    return _comb_call(partial, maskf)
